# baseline (speedup 1.0000x reference)
import dataclasses
import functools

import jax
import jax.numpy as jnp
from jax import lax
from jax.experimental import pallas as pl
from jax.experimental.pallas import tpu as pltpu
from jax.experimental.pallas import tpu_sc as plsc

N_NODES = 10000
D = 256
E = 160000
NC = 2
NS = 16
NW = NC * NS
B_PER_W = E // NW
W = 200
NCHUNK = B_PER_W // W
LANES = 16


def _dot_kernel(table_hbm, src_hbm, dst_hbm, out_hbm,
                sidx_v, didx_v, arows, brows, outv, sem_a, sem_b):
    wid = lax.axis_index("s") * NC + lax.axis_index("c")
    base = wid * B_PER_W
    pltpu.sync_copy(src_hbm.at[pl.ds(base, B_PER_W)], sidx_v)
    pltpu.sync_copy(dst_hbm.at[pl.ds(base, B_PER_W)], didx_v)

    lane = lax.iota(jnp.int32, LANES)
    last_lane = jnp.full((LANES,), LANES - 1, jnp.int32)

    def _edge_dot(w):
        nacc = 4
        parts = []
        for a in range(nacc):
            acc = (arows[w, pl.ds(a * LANES, LANES)]
                   * brows[w, pl.ds(a * LANES, LANES)])
            for c in range(a + nacc, D // LANES, nacc):
                acc = acc + (arows[w, pl.ds(c * LANES, LANES)]
                             * brows[w, pl.ds(c * LANES, LANES)])
            parts.append(acc)
        cs = jnp.cumsum((parts[0] + parts[1]) + (parts[2] + parts[3]))
        return lax.gather(
            cs, last_lane[:, None],
            lax.GatherDimensionNumbers(offset_dims=(),
                                       collapsed_slice_dims=(0,),
                                       start_index_map=(0,)),
            slice_sizes=(1,),
            mode=lax.GatherScatterMode.PROMISE_IN_BOUNDS)

    @pl.loop(0, NCHUNK)
    def _chunk(k):
        off = k * W
        cp_a = pltpu.async_copy(
            table_hbm.at[sidx_v.at[pl.ds(off, W)]], arows, sem_a)
        cp_b = pltpu.async_copy(
            table_hbm.at[didx_v.at[pl.ds(off, W)]], brows, sem_b)
        cp_a.wait()
        cp_b.wait()

        @pl.loop(0, W // LANES)
        def _group(g):
            res = jnp.zeros((LANES,), jnp.float32)
            for j in range(LANES):
                res = jnp.where(lane == j, _edge_dot(g * LANES + j), res)
            outv[pl.ds(g * LANES, LANES)] = res

        n_tail = W % LANES
        if n_tail:
            res = jnp.zeros((LANES,), jnp.float32)
            for j in range(n_tail):
                res = jnp.where(lane == j, _edge_dot((W // LANES) * LANES + j),
                                res)
            outv[pl.ds((W // LANES) * LANES, LANES)] = res

        pltpu.sync_copy(outv.at[pl.ds(0, W)], out_hbm.at[pl.ds(base + off, W)])


@jax.jit
def kernel(node_representations, edge_index):
    src = edge_index[0].astype(jnp.int32)
    dst = edge_index[1].astype(jnp.int32)

    mesh = plsc.VectorSubcoreMesh(core_axis_name="c", subcore_axis_name="s")
    cp = pltpu.CompilerParams()
    if "needs_layout_passes" in pltpu.CompilerParams.__dataclass_fields__:
        cp = dataclasses.replace(cp, needs_layout_passes=False)
    k = functools.partial(
        pl.kernel,
        mesh=mesh,
        compiler_params=cp,
        out_type=jax.ShapeDtypeStruct((E,), jnp.float32),
        scratch_types=[
            pltpu.VMEM((B_PER_W,), jnp.int32),
            pltpu.VMEM((B_PER_W,), jnp.int32),
            pltpu.VMEM((W, D), jnp.float32),
            pltpu.VMEM((W, D), jnp.float32),
            pltpu.VMEM((W + (-W) % LANES, ), jnp.float32),
            pltpu.SemaphoreType.DMA,
            pltpu.SemaphoreType.DMA,
        ],
    )(_dot_kernel)
    logits = k(node_representations, src, dst)
    return logits.reshape(E, 1)


# speedup vs baseline: 1.6226x; 1.6226x over previous
import dataclasses
import functools

import jax
import jax.numpy as jnp
from jax import lax
from jax.experimental import pallas as pl
from jax.experimental.pallas import tpu as pltpu
from jax.experimental.pallas import tpu_sc as plsc

N_NODES = 10000
D = 256
E = 160000
NC = 2
NS = 16
NW = NC * NS
B_PER_W = E // NW
W = 200
NCHUNK = B_PER_W // W
LANES = 16
BLANES = 32


def _dot_kernel(table_hbm, src_hbm, dst_hbm, out_hbm,
                sidx_v, didx_v, arows, brows, outv, sem_a, sem_b):
    wid = lax.axis_index("s") * NC + lax.axis_index("c")
    base = wid * B_PER_W
    pltpu.sync_copy(src_hbm.at[pl.ds(base, B_PER_W)], sidx_v)
    pltpu.sync_copy(dst_hbm.at[pl.ds(base, B_PER_W)], didx_v)

    lane = lax.iota(jnp.int32, LANES)
    last_lane = jnp.full((LANES,), LANES - 1, jnp.int32)

    def _edge_dot(w):
        acc_lo = acc_hi = None
        for c in range(D // BLANES):
            av = plsc.bitcast(arows[w, pl.ds(c * LANES, LANES)], jnp.bfloat16)
            bv = plsc.bitcast(brows[w, pl.ds(c * LANES, LANES)], jnp.bfloat16)
            prod = av * bv
            lo, hi = plsc.unpack(prod, format=plsc.PackFormat.INTERLEAVED)
            acc_lo = lo if acc_lo is None else acc_lo + lo
            acc_hi = hi if acc_hi is None else acc_hi + hi
        cs = jnp.cumsum(acc_lo + acc_hi)
        return lax.gather(
            cs, last_lane[:, None],
            lax.GatherDimensionNumbers(offset_dims=(),
                                       collapsed_slice_dims=(0,),
                                       start_index_map=(0,)),
            slice_sizes=(1,),
            mode=lax.GatherScatterMode.PROMISE_IN_BOUNDS)

    @pl.loop(0, NCHUNK)
    def _chunk(k):
        off = k * W
        cp_a = pltpu.async_copy(
            table_hbm.at[sidx_v.at[pl.ds(off, W)]], arows, sem_a)
        cp_b = pltpu.async_copy(
            table_hbm.at[didx_v.at[pl.ds(off, W)]], brows, sem_b)
        cp_a.wait()
        cp_b.wait()

        @pl.loop(0, W // LANES)
        def _group(g):
            res = jnp.zeros((LANES,), jnp.float32)
            for j in range(LANES):
                res = jnp.where(lane == j, _edge_dot(g * LANES + j), res)
            outv[pl.ds(g * LANES, LANES)] = res

        n_tail = W % LANES
        if n_tail:
            res = jnp.zeros((LANES,), jnp.float32)
            for j in range(n_tail):
                res = jnp.where(lane == j, _edge_dot((W // LANES) * LANES + j),
                                res)
            outv[pl.ds((W // LANES) * LANES, LANES)] = res

        pltpu.sync_copy(outv.at[pl.ds(0, W)], out_hbm.at[pl.ds(base + off, W)])


@jax.jit
def kernel(node_representations, edge_index):
    src = edge_index[0].astype(jnp.int32)
    dst = edge_index[1].astype(jnp.int32)
    table = lax.bitcast_convert_type(
        node_representations.astype(jnp.bfloat16).reshape(N_NODES, D // 2, 2),
        jnp.int32)

    mesh = plsc.VectorSubcoreMesh(core_axis_name="c", subcore_axis_name="s")
    cp = pltpu.CompilerParams()
    if "needs_layout_passes" in pltpu.CompilerParams.__dataclass_fields__:
        cp = dataclasses.replace(cp, needs_layout_passes=False)
    k = functools.partial(
        pl.kernel,
        mesh=mesh,
        compiler_params=cp,
        out_type=jax.ShapeDtypeStruct((E,), jnp.float32),
        scratch_types=[
            pltpu.VMEM((B_PER_W,), jnp.int32),
            pltpu.VMEM((B_PER_W,), jnp.int32),
            pltpu.VMEM((W, D // 2), jnp.int32),
            pltpu.VMEM((W, D // 2), jnp.int32),
            pltpu.VMEM((W + (-W) % LANES, ), jnp.float32),
            pltpu.SemaphoreType.DMA,
            pltpu.SemaphoreType.DMA,
        ],
    )(_dot_kernel)
    logits = k(table, src, dst)
    return logits.reshape(E, 1)


# speedup vs baseline: 2.0790x; 1.2813x over previous
import dataclasses
import functools

import jax
import jax.numpy as jnp
from jax import lax
from jax.experimental import pallas as pl
from jax.experimental.pallas import tpu as pltpu
from jax.experimental.pallas import tpu_sc as plsc

N_NODES = 10000
D = 256
E = 160000
NC = 2
NS = 16
NW = NC * NS
B_PER_W = E // NW
W = 200
NCHUNK = B_PER_W // W
LANES = 16
BLANES = 32


def _dot_kernel(table_hbm, src_hbm, dst_hbm, out_hbm,
                sidx_v, didx_v, arows0, brows0, arows1, brows1, outv,
                sem_a0, sem_b0, sem_a1, sem_b1):
    wid = lax.axis_index("s") * NC + lax.axis_index("c")
    base = wid * B_PER_W
    pltpu.sync_copy(src_hbm.at[pl.ds(base, B_PER_W)], sidx_v)
    pltpu.sync_copy(dst_hbm.at[pl.ds(base, B_PER_W)], didx_v)

    lane = lax.iota(jnp.int32, LANES)
    last_lane = jnp.full((LANES,), LANES - 1, jnp.int32)

    def _edge_dot(arows, brows, w):
        acc_lo = acc_hi = None
        for c in range(D // BLANES):
            av = plsc.bitcast(arows[w, pl.ds(c * LANES, LANES)], jnp.bfloat16)
            bv = plsc.bitcast(brows[w, pl.ds(c * LANES, LANES)], jnp.bfloat16)
            prod = av * bv
            lo, hi = plsc.unpack(prod, format=plsc.PackFormat.INTERLEAVED)
            acc_lo = lo if acc_lo is None else acc_lo + lo
            acc_hi = hi if acc_hi is None else acc_hi + hi
        cs = jnp.cumsum(acc_lo + acc_hi)
        return lax.gather(
            cs, last_lane[:, None],
            lax.GatherDimensionNumbers(offset_dims=(),
                                       collapsed_slice_dims=(0,),
                                       start_index_map=(0,)),
            slice_sizes=(1,),
            mode=lax.GatherScatterMode.PROMISE_IN_BOUNDS)

    def _issue(k, arows, brows, sem_a, sem_b):
        off = k * W
        cp_a = pltpu.async_copy(
            table_hbm.at[sidx_v.at[pl.ds(off, W)]], arows, sem_a)
        cp_b = pltpu.async_copy(
            table_hbm.at[didx_v.at[pl.ds(off, W)]], brows, sem_b)
        return cp_a, cp_b

    def _compute_resident(k, arows, brows):
        @pl.loop(0, W // LANES)
        def _group(g):
            res = jnp.zeros((LANES,), jnp.float32)
            for j in range(LANES):
                res = jnp.where(lane == j,
                                _edge_dot(arows, brows, g * LANES + j), res)
            outv[pl.ds(g * LANES, LANES)] = res

        n_tail = W % LANES
        if n_tail:
            res = jnp.zeros((LANES,), jnp.float32)
            for j in range(n_tail):
                res = jnp.where(lane == j,
                                _edge_dot(arows, brows,
                                          (W // LANES) * LANES + j), res)
            outv[pl.ds((W // LANES) * LANES, LANES)] = res

        pltpu.sync_copy(outv.at[pl.ds(0, W)],
                        out_hbm.at[pl.ds(base + k * W, W)])

    cp_a, cp_b = _issue(0, arows0, brows0, sem_a0, sem_b0)
    cp_a.wait()
    cp_b.wait()

    @pl.loop(0, NCHUNK - 1, step=2)
    def _pair(k):
        cp_a, cp_b = _issue(k + 1, arows1, brows1, sem_a1, sem_b1)
        _compute_resident(k, arows0, brows0)
        cp_a.wait()
        cp_b.wait()
        cp_a2, cp_b2 = _issue(k + 2, arows0, brows0, sem_a0, sem_b0)
        _compute_resident(k + 1, arows1, brows1)
        cp_a2.wait()
        cp_b2.wait()

    _compute_resident(NCHUNK - 1, arows0, brows0)


@jax.jit
def kernel(node_representations, edge_index):
    src = edge_index[0].astype(jnp.int32)
    dst = edge_index[1].astype(jnp.int32)
    table = lax.bitcast_convert_type(
        node_representations.astype(jnp.bfloat16).reshape(N_NODES, D // 2, 2),
        jnp.int32)

    mesh = plsc.VectorSubcoreMesh(core_axis_name="c", subcore_axis_name="s")
    cp = pltpu.CompilerParams()
    if "needs_layout_passes" in pltpu.CompilerParams.__dataclass_fields__:
        cp = dataclasses.replace(cp, needs_layout_passes=False)
    k = functools.partial(
        pl.kernel,
        mesh=mesh,
        compiler_params=cp,
        out_type=jax.ShapeDtypeStruct((E,), jnp.float32),
        scratch_types=[
            pltpu.VMEM((B_PER_W,), jnp.int32),
            pltpu.VMEM((B_PER_W,), jnp.int32),
            pltpu.VMEM((W, D // 2), jnp.int32),
            pltpu.VMEM((W, D // 2), jnp.int32),
            pltpu.VMEM((W, D // 2), jnp.int32),
            pltpu.VMEM((W, D // 2), jnp.int32),
            pltpu.VMEM((W + (-W) % LANES, ), jnp.float32),
            pltpu.SemaphoreType.DMA,
            pltpu.SemaphoreType.DMA,
            pltpu.SemaphoreType.DMA,
            pltpu.SemaphoreType.DMA,
        ],
    )(_dot_kernel)
    logits = k(table, src, dst)
    return logits.reshape(E, 1)


# speedup vs baseline: 3.6726x; 1.7665x over previous
import dataclasses
import functools

import jax
import jax.numpy as jnp
from jax import lax
from jax.experimental import pallas as pl
from jax.experimental.pallas import tpu as pltpu
from jax.experimental.pallas import tpu_sc as plsc

N_NODES = 10000
D = 256
E = 160000
NC = 2
NS = 16
NW = NC * NS
B_PER_W = E // NW
W = 200
NCHUNK = B_PER_W // W
LANES = 16
BLANES = 32


def _dot_kernel(table_hbm, ei_hbm, out_hbm,
                sidx_v, didx_v, arows0, brows0, arows1, brows1, outv,
                sem_a0, sem_b0, sem_a1, sem_b1):
    wid = lax.axis_index("s") * NC + lax.axis_index("c")
    base = wid * B_PER_W
    pltpu.sync_copy(ei_hbm.at[pl.ds(base, B_PER_W)], sidx_v)
    pltpu.sync_copy(ei_hbm.at[pl.ds(E + base, B_PER_W)], didx_v)

    lane = lax.iota(jnp.int32, LANES)
    last_lane = jnp.full((LANES,), LANES - 1, jnp.int32)

    def _edge_dot(arows, brows, w):
        acc_lo = acc_hi = None
        for c in range(D // BLANES):
            av = plsc.bitcast(arows[w, pl.ds(c * LANES, LANES)], jnp.bfloat16)
            bv = plsc.bitcast(brows[w, pl.ds(c * LANES, LANES)], jnp.bfloat16)
            prod = av * bv
            lo, hi = plsc.unpack(prod, format=plsc.PackFormat.INTERLEAVED)
            acc_lo = lo if acc_lo is None else acc_lo + lo
            acc_hi = hi if acc_hi is None else acc_hi + hi
        cs = jnp.cumsum(acc_lo + acc_hi)
        return lax.gather(
            cs, last_lane[:, None],
            lax.GatherDimensionNumbers(offset_dims=(),
                                       collapsed_slice_dims=(0,),
                                       start_index_map=(0,)),
            slice_sizes=(1,),
            mode=lax.GatherScatterMode.PROMISE_IN_BOUNDS)

    def _issue(k, arows, brows, sem_a, sem_b):
        off = k * W
        cp_a = pltpu.async_copy(
            table_hbm.at[sidx_v.at[pl.ds(off, W)]], arows, sem_a)
        cp_b = pltpu.async_copy(
            table_hbm.at[didx_v.at[pl.ds(off, W)]], brows, sem_b)
        return cp_a, cp_b

    def _compute_resident(k, arows, brows):
        @pl.loop(0, W // LANES)
        def _group(g):
            res = jnp.zeros((LANES,), jnp.float32)
            for j in range(LANES):
                res = jnp.where(lane == j,
                                _edge_dot(arows, brows, g * LANES + j), res)
            outv[pl.ds(g * LANES, LANES)] = res

        n_tail = W % LANES
        if n_tail:
            res = jnp.zeros((LANES,), jnp.float32)
            for j in range(n_tail):
                res = jnp.where(lane == j,
                                _edge_dot(arows, brows,
                                          (W // LANES) * LANES + j), res)
            outv[pl.ds((W // LANES) * LANES, LANES)] = res

        pltpu.sync_copy(outv.at[pl.ds(0, W)],
                        out_hbm.at[pl.ds(base + k * W, W)])

    cp_a, cp_b = _issue(0, arows0, brows0, sem_a0, sem_b0)
    cp_a.wait()
    cp_b.wait()

    @pl.loop(0, NCHUNK - 1, step=2)
    def _pair(k):
        cp_a, cp_b = _issue(k + 1, arows1, brows1, sem_a1, sem_b1)
        _compute_resident(k, arows0, brows0)
        cp_a.wait()
        cp_b.wait()
        cp_a2, cp_b2 = _issue(k + 2, arows0, brows0, sem_a0, sem_b0)
        _compute_resident(k + 1, arows1, brows1)
        cp_a2.wait()
        cp_b2.wait()

    _compute_resident(NCHUNK - 1, arows0, brows0)


@jax.jit
def kernel(node_representations, edge_index):
    ei = edge_index.astype(jnp.int32).reshape(2 * E)
    lo = lax.bitcast_convert_type(
        node_representations[:, :D // 2].astype(jnp.bfloat16),
        jnp.uint16).astype(jnp.uint32)
    hi = lax.bitcast_convert_type(
        node_representations[:, D // 2:].astype(jnp.bfloat16),
        jnp.uint16).astype(jnp.uint32)
    table = lax.bitcast_convert_type(lo | (hi << 16), jnp.int32)

    mesh = plsc.VectorSubcoreMesh(core_axis_name="c", subcore_axis_name="s")
    cp = pltpu.CompilerParams()
    if "needs_layout_passes" in pltpu.CompilerParams.__dataclass_fields__:
        cp = dataclasses.replace(cp, needs_layout_passes=False)
    k = functools.partial(
        pl.kernel,
        mesh=mesh,
        compiler_params=cp,
        out_type=jax.ShapeDtypeStruct((E,), jnp.float32),
        scratch_types=[
            pltpu.VMEM((B_PER_W,), jnp.int32),
            pltpu.VMEM((B_PER_W,), jnp.int32),
            pltpu.VMEM((W, D // 2), jnp.int32),
            pltpu.VMEM((W, D // 2), jnp.int32),
            pltpu.VMEM((W, D // 2), jnp.int32),
            pltpu.VMEM((W, D // 2), jnp.int32),
            pltpu.VMEM((W + (-W) % LANES, ), jnp.float32),
            pltpu.SemaphoreType.DMA,
            pltpu.SemaphoreType.DMA,
            pltpu.SemaphoreType.DMA,
            pltpu.SemaphoreType.DMA,
        ],
    )(_dot_kernel)
    logits = k(table, ei)
    return logits.reshape(E, 1)
